# baseline (speedup 1.0000x reference)
import functools

import jax
import jax.numpy as jnp
from jax import lax
from jax.experimental import pallas as pl
from jax.experimental.pallas import tpu as pltpu
from jax.experimental.pallas import tpu_sc as plsc


def _make_sc_gather(V, D, B, n_workers):
    assert B % n_workers == 0
    b_per_w = B // n_workers
    C = 400
    NS = 4
    LOOKAHEAD = 2
    assert b_per_w % C == 0
    n_chunks = b_per_w // C

    mesh = plsc.VectorSubcoreMesh(core_axis_name="c", subcore_axis_name="s")

    @functools.partial(
        pl.kernel,
        mesh=mesh,
        compiler_params=pltpu.CompilerParams(use_tc_tiling_on_sc=False),
        out_type=jax.ShapeDtypeStruct((B, D), jnp.float32),
        scratch_types=[
            pltpu.VMEM((b_per_w,), jnp.int32),
            [pltpu.VMEM((C, D), jnp.float32) for _ in range(NS)],
            [pltpu.SemaphoreType.DMA for _ in range(NS)],
            [pltpu.SemaphoreType.DMA for _ in range(NS)],
        ],
    )
    def gather_kernel(table_hbm, idx_hbm, out_hbm, idx_v, rows, gsems, ssems):
        n_cores = lax.axis_size("c")
        wid = lax.axis_index("s") * n_cores + lax.axis_index("c")
        base = wid * b_per_w

        pltpu.sync_copy(idx_hbm.at[pl.ds(base, b_per_w)], idx_v)

        def gather(c):
            b = c % NS
            pltpu.async_copy(
                table_hbm.at[idx_v.at[pl.ds(c * C, C)]], rows[b], gsems[b]
            )

        def wait_gather(c):
            b = c % NS
            pltpu.make_async_copy(
                table_hbm.at[idx_v.at[pl.ds(c * C, C)]], rows[b], gsems[b]
            ).wait()

        def scatter(c):
            b = c % NS
            pltpu.async_copy(rows[b], out_hbm.at[pl.ds(base + c * C, C)], ssems[b])

        def wait_scatter(c):
            b = c % NS
            pltpu.make_async_copy(
                rows[b], out_hbm.at[pl.ds(base + c * C, C)], ssems[b]
            ).wait()

        for c in range(min(LOOKAHEAD, n_chunks)):
            gather(c)
        for c in range(n_chunks):
            wait_gather(c)
            scatter(c)
            f = c + LOOKAHEAD
            if f < n_chunks:
                if f >= NS:
                    wait_scatter(f - NS)
                gather(f)
        for c in range(max(0, n_chunks - NS), n_chunks):
            wait_scatter(c)

    return gather_kernel


def kernel(captions, table):
    B, S = captions.shape
    V, D = table.shape
    flat_idx = captions.reshape(B * S).astype(jnp.int32)
    info = plsc.get_sparse_core_info()
    n_workers = info.num_cores * info.num_subcores
    out = _make_sc_gather(V, D, B * S, n_workers)(table, flat_idx)
    return out.reshape(B, S, D)
